# baseline (speedup 1.0000x reference)
import functools

import jax
import jax.numpy as jnp
from jax import lax
from jax.experimental import pallas as pl
from jax.experimental.pallas import tpu as pltpu
from jax.experimental.pallas import tpu_sc as plsc

_L = 16


def _rsqrt_vec(x):
    t = x
    scale = jnp.full((_L,), 1.0, jnp.float32)
    for _ in range(8):
        big = t > 2.0
        t = jnp.where(big, t * 0.25, t)
        scale = jnp.where(big, scale * 0.5, scale)
    for _ in range(8):
        small = t < 0.5
        t = jnp.where(small, t * 4.0, t)
        scale = jnp.where(small, scale * 2.0, scale)
    y = jnp.full((_L,), 1.0, jnp.float32)
    for _ in range(5):
        y = y * (jnp.float32(1.5) - jnp.float32(0.5) * t * y * y)
    return y * scale


def kernel(inp, vx, gx, vy, gy):
    b = inp.shape[0]
    H, D = vx.shape
    W = vy.shape[0]
    rows, width = W * H, 2 * D
    nj = D // _L

    info = plsc.get_sparse_core_info()
    NW = info.num_cores * info.num_subcores
    rpw = rows // NW

    @functools.partial(
        pl.kernel,
        mesh=plsc.VectorSubcoreMesh(core_axis_name="c", subcore_axis_name="s"),
        out_type=jax.ShapeDtypeStruct((b, rows, width), jnp.float32),
        scratch_types=[
            pltpu.VMEM((H, D), jnp.float32),
            pltpu.VMEM((D,), jnp.float32),
            pltpu.VMEM((H + _L,), jnp.float32),
            pltpu.VMEM((W + _L,), jnp.float32),
            pltpu.VMEM((rpw, width), jnp.float32),
            pltpu.SemaphoreType.DMA,
        ],
    )
    def sc_posemb(vx_hbm, gx_hbm, vy_hbm, gy_hbm, out_hbm,
                  vxv, vyr, gxv, gyv, chunk, sem):
        wid = lax.axis_index("s") * info.num_cores + lax.axis_index("c")
        base = wid * rpw

        pltpu.sync_copy(vx_hbm, vxv)
        pltpu.sync_copy(gx_hbm, gxv.at[pl.ds(0, H)])
        pltpu.sync_copy(vy_hbm.at[wid], vyr)
        pltpu.sync_copy(gy_hbm, gyv.at[pl.ds(0, W)])

        lanes = lax.iota(jnp.int32, _L)
        lane0 = jnp.zeros((_L,), jnp.int32)

        def allsum(v):
            for k in (8, 4, 2, 1):
                v = v + v.at[lanes ^ k].get(mode="promise_in_bounds")
            return v

        def splat0(v):
            return v.at[lane0].get(mode="promise_in_bounds")

        acc = jnp.zeros((_L,), jnp.float32)
        for jc in range(nj):
            v = vyr[pl.ds(jc * _L, _L)]
            acc = acc + v * v
        syv = _rsqrt_vec(allsum(acc)) * splat0(gyv[pl.ds(wid, _L)])
        for jc in range(nj):
            s = pl.ds(jc * _L, _L)
            vyr[s] = vyr[s] * syv

        def build_row(h, carry):
            acc = jnp.zeros((_L,), jnp.float32)
            for jc in range(nj):
                v = vxv[h, pl.ds(jc * _L, _L)]
                acc = acc + v * v
            sxv = _rsqrt_vec(allsum(acc)) * splat0(gxv[pl.ds(h, _L)])
            for jc in range(nj):
                chunk[h, pl.ds(jc * _L, _L)] = vxv[h, pl.ds(jc * _L, _L)] * sxv
                chunk[h, pl.ds(D + jc * _L, _L)] = vyr[pl.ds(jc * _L, _L)]
            return carry
        lax.fori_loop(0, rpw, build_row, 0)

        descs = [
            pltpu.async_copy(chunk, out_hbm.at[i, pl.ds(base, rpw)], sem)
            for i in range(b)
        ]
        for d in descs:
            d.wait()

    return sc_posemb(vx, gx.reshape(H), vy, gy.reshape(W))


# speedup vs baseline: 1.0304x; 1.0304x over previous
import functools

import jax
import jax.numpy as jnp
from jax import lax
from jax.experimental import pallas as pl
from jax.experimental.pallas import tpu as pltpu
from jax.experimental.pallas import tpu_sc as plsc

_L = 16


def _rsqrt_vec(x):
    t = x
    scale = jnp.full((_L,), 1.0, jnp.float32)
    for _ in range(8):
        big = t > 2.0
        t = jnp.where(big, t * 0.25, t)
        scale = jnp.where(big, scale * 0.5, scale)
    for _ in range(8):
        small = t < 0.5
        t = jnp.where(small, t * 4.0, t)
        scale = jnp.where(small, scale * 2.0, scale)
    y = jnp.full((_L,), 1.0, jnp.float32)
    for _ in range(5):
        y = y * (jnp.float32(1.5) - jnp.float32(0.5) * t * y * y)
    return y * scale


def kernel(inp, vx, gx, vy, gy):
    b = inp.shape[0]
    H, D = vx.shape
    W = vy.shape[0]
    rows, width = W * H, 2 * D
    nj = D // _L

    info = plsc.get_sparse_core_info()
    NW = info.num_cores * info.num_subcores
    rpw = rows // NW

    @functools.partial(
        pl.kernel,
        mesh=plsc.VectorSubcoreMesh(core_axis_name="c", subcore_axis_name="s"),
        out_type=jax.ShapeDtypeStruct((b, rows, width), jnp.float32),
        scratch_types=[
            pltpu.VMEM((H, D), jnp.float32),
            pltpu.VMEM((D,), jnp.float32),
            pltpu.VMEM((H + _L,), jnp.float32),
            pltpu.VMEM((W + _L,), jnp.float32),
            pltpu.VMEM((rpw, width), jnp.float32),
            pltpu.SemaphoreType.DMA,
        ],
    )
    def sc_posemb(vx_hbm, gx_hbm, vy_hbm, gy_hbm, out_hbm,
                  vxv, vyr, gxv, gyv, chunk, sem):
        wid = lax.axis_index("s") * info.num_cores + lax.axis_index("c")
        base = wid * rpw

        pltpu.sync_copy(vx_hbm, vxv)
        pltpu.sync_copy(gx_hbm, gxv.at[pl.ds(0, H)])
        pltpu.sync_copy(vy_hbm.at[wid], vyr)
        pltpu.sync_copy(gy_hbm, gyv.at[pl.ds(0, W)])

        lanes = lax.iota(jnp.int32, _L)
        lane0 = jnp.zeros((_L,), jnp.int32)

        def allsum(v):
            for k in (8, 4, 2, 1):
                v = v + v.at[lanes ^ k].get(mode="promise_in_bounds")
            return v

        def splat0(v):
            return v.at[lane0].get(mode="promise_in_bounds")

        def row_ss(ref, r):
            acc = ref[r, pl.ds(0, _L)] * ref[r, pl.ds(0, _L)]
            for jc in range(1, nj):
                v = ref[r, pl.ds(jc * _L, _L)]
                acc = acc + v * v
            return allsum(acc)

        packs = []
        for g in range(H // _L):
            p = jnp.zeros((_L,), jnp.float32)
            for r in range(_L):
                p = jnp.where(lanes == r, row_ss(vxv, g * _L + r), p)
            packs.append(_rsqrt_vec(p) * gxv[pl.ds(g * _L, _L)])

        accy = vyr[pl.ds(0, _L)] * vyr[pl.ds(0, _L)]
        for jc in range(1, nj):
            v = vyr[pl.ds(jc * _L, _L)]
            accy = accy + v * v
        syv = _rsqrt_vec(allsum(accy)) * splat0(gyv[pl.ds(wid, _L)])
        yrow = [vyr[pl.ds(jc * _L, _L)] * syv for jc in range(nj)]

        for h in range(rpw):
            sxv = packs[h // _L].at[
                jnp.full((_L,), h % _L, jnp.int32)].get(mode="promise_in_bounds")
            for jc in range(nj):
                chunk[h, pl.ds(jc * _L, _L)] = vxv[h, pl.ds(jc * _L, _L)] * sxv
                chunk[h, pl.ds(D + jc * _L, _L)] = yrow[jc]

        descs = [
            pltpu.async_copy(chunk, out_hbm.at[i, pl.ds(base, rpw)], sem)
            for i in range(b)
        ]
        for d in descs:
            d.wait()

    return sc_posemb(vx, gx.reshape(H), vy, gy.reshape(W))


# speedup vs baseline: 1.0620x; 1.0306x over previous
import functools

import jax
import jax.numpy as jnp
from jax import lax
from jax.experimental import pallas as pl
from jax.experimental.pallas import tpu as pltpu
from jax.experimental.pallas import tpu_sc as plsc


def _pattern_body(vx_ref, gx_ref, vy_ref, gy_ref, out_ref):
    H = vx_ref.shape[0]
    W = vy_ref.shape[0]
    vx = vx_ref[...]
    wx = gx_ref[...] * vx * jax.lax.rsqrt(jnp.sum(vx * vx, axis=1, keepdims=True))
    vy = vy_ref[...]
    wy = gy_ref[...] * vy * jax.lax.rsqrt(jnp.sum(vy * vy, axis=1, keepdims=True))
    xblock = jnp.tile(wx, (W, 1))
    yblock = jnp.repeat(wy, H, axis=0)
    out_ref[...] = jnp.concatenate([xblock, yblock], axis=1)


def kernel(inp, vx, gx, vy, gy):
    b = inp.shape[0]
    H, D = vx.shape
    W = vy.shape[0]
    rows, width = W * H, 2 * D

    full = lambda s: pl.BlockSpec(s, lambda: (0,) * len(s))
    pattern = pl.pallas_call(
        _pattern_body,
        in_specs=[full((H, D)), full((H, 1)), full((W, D)), full((W, 1))],
        out_specs=full((rows, width)),
        out_shape=jax.ShapeDtypeStruct((rows, width), jnp.float32),
    )(vx, gx, vy, gy)

    info = plsc.get_sparse_core_info()
    NC, NS = info.num_cores, info.num_subcores
    NW = NC * NS
    rpw = rows // NW

    @functools.partial(
        pl.kernel,
        mesh=plsc.VectorSubcoreMesh(core_axis_name="c", subcore_axis_name="s"),
        out_type=jax.ShapeDtypeStruct((b, rows, width), jnp.float32),
        scratch_types=[
            pltpu.VMEM((rpw, width), jnp.float32),
            pltpu.VMEM_SHARED((rows, width), jnp.float32),
            pltpu.SemaphoreType.DMA,
            pltpu.SemaphoreType.DMA,
        ],
    )
    def sc_broadcast(pattern_hbm, out_hbm, chunk, shared, sem, sem2):
        sid = lax.axis_index("s")
        wid = sid * NC + lax.axis_index("c")
        base = wid * rpw
        pltpu.sync_copy(pattern_hbm.at[pl.ds(base, rpw)], chunk)
        @pl.when(sid == 0)
        def _():
            pltpu.sync_copy(pattern_hbm, shared)
        plsc.subcore_barrier()
        descs = []
        for i in range(b):
            if i % 2 == 0:
                descs.append(
                    pltpu.async_copy(chunk, out_hbm.at[i, pl.ds(base, rpw)], sem))
            else:
                descs.append(
                    pltpu.async_copy(shared.at[pl.ds(base, rpw)],
                                     out_hbm.at[i, pl.ds(base, rpw)], sem2))
        for d in descs:
            d.wait()

    return sc_broadcast(pattern)


# speedup vs baseline: 1.1542x; 1.0869x over previous
import functools

import jax
import jax.numpy as jnp
from jax import lax
from jax.experimental import pallas as pl
from jax.experimental.pallas import tpu as pltpu
from jax.experimental.pallas import tpu_sc as plsc


def _pattern_body(vx_ref, g_ref, vy_ref, out_ref):
    H = vx_ref.shape[0]
    W = vy_ref.shape[0]
    vx = vx_ref[...]
    gx = g_ref[:, 0:1]
    gy = g_ref[:, 1:2]
    wx = gx * vx * jax.lax.rsqrt(jnp.sum(vx * vx, axis=1, keepdims=True))
    vy = vy_ref[...]
    wy = gy * vy * jax.lax.rsqrt(jnp.sum(vy * vy, axis=1, keepdims=True))
    xblock = jnp.tile(wx, (W, 1))
    yblock = jnp.repeat(wy, H, axis=0)
    out_ref[...] = jnp.concatenate([xblock, yblock], axis=1)


def kernel(inp, vx, gx, vy, gy):
    b = inp.shape[0]
    H, D = vx.shape
    W = vy.shape[0]
    rows, width = W * H, 2 * D
    g2 = jnp.concatenate([gx, gy], axis=1)

    full = lambda s: pl.BlockSpec(s, lambda: (0,) * len(s))
    pattern = pl.pallas_call(
        _pattern_body,
        in_specs=[full((H, D)), full((H, 2)), full((W, D))],
        out_specs=full((rows, width)),
        out_shape=jax.ShapeDtypeStruct((rows, width), jnp.float32),
    )(vx, g2, vy)

    info = plsc.get_sparse_core_info()
    NW = info.num_cores * info.num_subcores
    rpw = rows // NW

    @functools.partial(
        pl.kernel,
        mesh=plsc.VectorSubcoreMesh(core_axis_name="c", subcore_axis_name="s"),
        out_type=jax.ShapeDtypeStruct((b, rows, width), jnp.float32),
        scratch_types=[
            pltpu.VMEM((rpw, width), jnp.float32),
            pltpu.SemaphoreType.DMA,
        ],
    )
    def sc_broadcast(pattern_hbm, out_hbm, chunk, sem):
        wid = lax.axis_index("s") * info.num_cores + lax.axis_index("c")
        base = wid * rpw
        pltpu.sync_copy(pattern_hbm.at[pl.ds(base, rpw)], chunk)
        descs = [
            pltpu.async_copy(chunk, out_hbm.at[i, pl.ds(base, rpw)], sem)
            for i in range(b)
        ]
        for d in descs:
            d.wait()

    return sc_broadcast(pattern)
